# baseline (speedup 1.0000x reference)
import functools

import jax
import jax.numpy as jnp
from jax import lax
from jax.experimental import pallas as pl
from jax.experimental.pallas import tpu as pltpu
from jax.experimental.pallas import tpu_sc as plsc

N = 10000
E = 320000
D = 128

_MIN_NORM = 1e-15
_MAXNORM = 1.0 - 4e-3

_NC = 2
_NS = 16
_NW = _NC * _NS
_EPW = E // _NW
_CHUNK = 80
_NCHUNK = _EPW // _CHUNK
_NPAD = 10240
_RPT = _NPAD // _NS
_ZROWS = 16


def _norm(x):
    return jnp.sqrt(jnp.sum(x * x, axis=-1, keepdims=True))


def _artanh(x):
    x = jnp.clip(x, -1.0 + 1e-7, 1.0 - 1e-7)
    return 0.5 * jnp.log((1.0 + x) / (1.0 - x))


def _tanh(x):
    return jnp.tanh(jnp.clip(x, -15.0, 15.0))


def _proj(x):
    n = jnp.maximum(_norm(x), _MIN_NORM)
    return jnp.where(n > _MAXNORM, x / n * _MAXNORM, x)


def _expmap0(u):
    n = jnp.maximum(_norm(u), _MIN_NORM)
    return _tanh(n) * u / n


def _logmap0(p):
    n = jnp.maximum(_norm(p), _MIN_NORM)
    return p / n * _artanh(n)


def _mobius_add(x, y):
    x2 = jnp.sum(x * x, axis=-1, keepdims=True)
    y2 = jnp.sum(y * y, axis=-1, keepdims=True)
    xy = jnp.sum(x * y, axis=-1, keepdims=True)
    num = (1.0 + 2.0 * xy + y2) * x + (1.0 - x2) * y
    denom = 1.0 + 2.0 * xy + x2 * y2
    return num / jnp.maximum(denom, _MIN_NORM)


def _mobius_matvec(W, x):
    xn = jnp.maximum(_norm(x), _MIN_NORM)
    mx = lax.dot_general(x, W, (((1,), (1,)), ((), ())),
                         precision=lax.Precision.HIGHEST)
    mxn = jnp.maximum(_norm(mx), _MIN_NORM)
    res = _tanh(mxn / xn * _artanh(xn)) * mx / mxn
    cond = jnp.all(mx == 0, axis=-1, keepdims=True)
    return jnp.where(cond, jnp.zeros_like(res), res)


def _hyp_linear_to_tangent(xh, W, b):
    res = _proj(_mobius_matvec(W, xh))
    hb = _proj(_expmap0(b))
    res = _proj(_mobius_add(res, hb))
    return _logmap0(res)


_BLK = 1000


def _k1_body(x_ref, w_ref, b_ref, o_ref):
    xh = _proj(_expmap0(x_ref[...]))
    o_ref[...] = _hyp_linear_to_tangent(xh, w_ref[...], b_ref[...])


def _k2_body(p_ref, w_ref, b_ref, o_ref):
    agg = p_ref[0] + p_ref[1]
    h = _proj(_expmap0(agg))
    xt = jax.nn.relu(_logmap0(h))
    xh = _proj(_expmap0(xt))
    o_ref[...] = _hyp_linear_to_tangent(xh, w_ref[...], b_ref[...])


def _k3_body(p_ref, o_ref):
    agg = p_ref[0] + p_ref[1]
    h = _proj(_expmap0(agg))
    xt = jax.nn.relu(_logmap0(h))
    o_ref[...] = _proj(_expmap0(xt))


def _dense_pre(x, W, b):
    return pl.pallas_call(
        _k1_body,
        grid=(N // _BLK,),
        in_specs=[
            pl.BlockSpec((_BLK, D), lambda i: (i, 0)),
            pl.BlockSpec((D, D), lambda i: (0, 0)),
            pl.BlockSpec((1, D), lambda i: (0, 0)),
        ],
        out_specs=pl.BlockSpec((_BLK, D), lambda i: (i, 0)),
        out_shape=jax.ShapeDtypeStruct((N, D), jnp.float32),
    )(x, W, b.reshape(1, D))


def _dense_mid(parts, W, b):
    return pl.pallas_call(
        _k2_body,
        grid=(N // _BLK,),
        in_specs=[
            pl.BlockSpec((2, _BLK, D), lambda i: (0, i, 0)),
            pl.BlockSpec((D, D), lambda i: (0, 0)),
            pl.BlockSpec((1, D), lambda i: (0, 0)),
        ],
        out_specs=pl.BlockSpec((_BLK, D), lambda i: (i, 0)),
        out_shape=jax.ShapeDtypeStruct((N, D), jnp.float32),
    )(parts, W, b.reshape(1, D))


def _dense_post(parts):
    return pl.pallas_call(
        _k3_body,
        grid=(N // _BLK,),
        in_specs=[
            pl.BlockSpec((2, _BLK, D), lambda i: (0, i, 0)),
        ],
        out_specs=pl.BlockSpec((_BLK, D), lambda i: (i, 0)),
        out_shape=jax.ShapeDtypeStruct((N, D), jnp.float32),
    )(parts)


def _lane_bcast(vec, j):
    return lax.gather(
        vec, jnp.full((16, 1), j, jnp.int32),
        lax.GatherDimensionNumbers(
            offset_dims=(), collapsed_slice_dims=(0,), start_index_map=(0,)),
        (1,), mode=lax.GatherScatterMode.PROMISE_IN_BOUNDS)


def _sc_body(xt_hbm, src_hbm, dst_hbm, w_hbm, out_hbm,
             src_v, dst_v, w_v, rows_v, zbuf_v, acc_sh,
             sem_i0, sem_i1, sem_i2, sem_i3,
             sem_g0, sem_g1, sem_g2, sem_g3,
             sem_a0, sem_a1, sem_a2, sem_a3):
    sem_i = (sem_i0, sem_i1, sem_i2, sem_i3)
    sem_g = (sem_g0, sem_g1, sem_g2, sem_g3)
    sem_a = (sem_a0, sem_a1, sem_a2, sem_a3)
    cid = lax.axis_index("c")
    sid = lax.axis_index("s")
    wid = cid * _NS + sid
    ebase = wid * _EPW
    row0 = sid * _RPT

    def issue_idx(c, b):
        base = ebase + c * _CHUNK
        pltpu.async_copy(src_hbm.at[pl.ds(base, _CHUNK)], src_v.at[b], sem_i[b])
        pltpu.async_copy(dst_hbm.at[pl.ds(base, _CHUNK)], dst_v.at[b], sem_i[b])
        pltpu.async_copy(w_hbm.at[pl.ds(base, _CHUNK)], w_v.at[b], sem_i[b])

    def wait_idx(b):
        pltpu.make_async_copy(
            src_hbm.at[pl.ds(0, _CHUNK)], src_v.at[b], sem_i[b]).wait()
        pltpu.make_async_copy(
            dst_hbm.at[pl.ds(0, _CHUNK)], dst_v.at[b], sem_i[b]).wait()
        pltpu.make_async_copy(
            w_hbm.at[pl.ds(0, _CHUNK)], w_v.at[b], sem_i[b]).wait()

    def issue_gather(b):
        pltpu.async_copy(xt_hbm.at[src_v.at[b]], rows_v.at[b], sem_g[b])

    def wait_gather(b):
        pltpu.make_async_copy(
            xt_hbm.at[src_v.at[b]], rows_v.at[b], sem_g[b]).wait()

    def issue_scatter(b):
        pltpu.async_copy(rows_v.at[b], acc_sh.at[dst_v.at[b]], sem_a[b],
                         add=True)

    def wait_scatter(b):
        pltpu.make_async_copy(
            rows_v.at[b], acc_sh.at[dst_v.at[b]], sem_a[b]).wait()

    def scale(b):
        @pl.loop(0, _CHUNK // 16)
        def _(g):
            wv = w_v[b, pl.ds(g * 16, 16)]

            @pl.loop(0, 16)
            def _(j):
                wb = _lane_bcast(wv, j)
                e = g * 16 + j
                for f in range(D // 16):
                    sl = pl.ds(f * 16, 16)
                    rows_v[b, e, sl] = rows_v[b, e, sl] * wb

    for b in range(4):
        issue_idx(b, b)

    @pl.loop(0, _ZROWS)
    def _(r):
        zero16 = jnp.zeros((16,), jnp.float32)
        for f in range(D // 16):
            zbuf_v[r, pl.ds(f * 16, 16)] = zero16

    for k in range(_RPT // _ZROWS):
        pltpu.sync_copy(zbuf_v, acc_sh.at[pl.ds(row0 + k * _ZROWS, _ZROWS)])

    wait_idx(0)
    issue_gather(0)
    wait_idx(1)
    issue_gather(1)
    plsc.subcore_barrier()

    @pl.loop(0, _NCHUNK // 4)
    def _(h):
        for r in range(4):
            c = 4 * h + r

            def lookahead():
                wait_scatter((r + 3) % 4)
                issue_idx(c + 3, (r + 3) % 4)
                wait_idx((r + 2) % 4)
                issue_gather((r + 2) % 4)

            wait_gather(r)
            if r == 0:
                @pl.when(h >= 1)
                def _():
                    lookahead()
                @pl.when(h < 1)
                def _():
                    wait_idx(2)
                    issue_gather(2)
            elif r <= 1:
                lookahead()
            else:
                @pl.when(h <= _NCHUNK // 4 - 2)
                def _():
                    lookahead()
                @pl.when(h > _NCHUNK // 4 - 2)
                def _():
                    if r == 2:
                        wait_scatter((r + 3) % 4)
                        wait_idx((r + 2) % 4)
                        issue_gather((r + 2) % 4)
                    else:
                        wait_scatter((r + 3) % 4)
            scale(r)
            issue_scatter(r)

    wait_gather(0)
    scale(0)
    issue_scatter(0)
    wait_scatter(3)
    wait_scatter(0)
    plsc.subcore_barrier()

    drains = []
    for k in range(_RPT // 128):
        r = row0 + k * 128
        drains.append(pltpu.async_copy(
            acc_sh.at[pl.ds(r, 128)], out_hbm.at[cid, pl.ds(r, 128)],
            sem_g0))
    for d in drains:
        d.wait()


def _sc_aggregate(x_t, src, dst, w):
    kern = pl.kernel(
        _sc_body,
        out_type=jax.ShapeDtypeStruct((_NC, _NPAD, D), jnp.float32),
        mesh=plsc.VectorSubcoreMesh(core_axis_name="c", subcore_axis_name="s"),
        scratch_types=[
            pltpu.VMEM((4, _CHUNK), jnp.int32),
            pltpu.VMEM((4, _CHUNK), jnp.int32),
            pltpu.VMEM((4, _CHUNK), jnp.float32),
            pltpu.VMEM((4, _CHUNK, D), jnp.float32),
            pltpu.VMEM((_ZROWS, D), jnp.float32),
            pltpu.VMEM_SHARED((_NPAD, D), jnp.float32),
        ] + [pltpu.SemaphoreType.DMA] * 12,
    )
    return kern(x_t, src, dst, w)


def kernel(x, edge_index, edge_weight, W1, b1, W2, b2):
    src = edge_index[0]
    dst = edge_index[1]
    xt1 = _dense_pre(x, W1, b1)
    parts1 = _sc_aggregate(xt1, src, dst, edge_weight)
    xt2 = _dense_mid(parts1, W2, b2)
    parts2 = _sc_aggregate(xt2, src, dst, edge_weight)
    return _dense_post(parts2)


# speedup vs baseline: 1.0066x; 1.0066x over previous
import functools

import jax
import jax.numpy as jnp
from jax import lax
from jax.experimental import pallas as pl
from jax.experimental.pallas import tpu as pltpu
from jax.experimental.pallas import tpu_sc as plsc

N = 10000
E = 320000
D = 128

_MIN_NORM = 1e-15
_MAXNORM = 1.0 - 4e-3

_NC = 2
_NS = 16
_NW = _NC * _NS
_EPW = E // _NW
_CHUNK = 80
_NCHUNK = _EPW // _CHUNK
_NPAD = 10240
_RPT = _NPAD // _NS
_ZROWS = 16


def _norm(x):
    return jnp.sqrt(jnp.sum(x * x, axis=-1, keepdims=True))


def _artanh(x):
    x = jnp.clip(x, -1.0 + 1e-7, 1.0 - 1e-7)
    return 0.5 * jnp.log((1.0 + x) / (1.0 - x))


def _tanh(x):
    return jnp.tanh(jnp.clip(x, -15.0, 15.0))


def _proj(x):
    n = jnp.maximum(_norm(x), _MIN_NORM)
    return jnp.where(n > _MAXNORM, x / n * _MAXNORM, x)


def _expmap0(u):
    n = jnp.maximum(_norm(u), _MIN_NORM)
    return _tanh(n) * u / n


def _logmap0(p):
    n = jnp.maximum(_norm(p), _MIN_NORM)
    return p / n * _artanh(n)


def _mobius_add(x, y):
    x2 = jnp.sum(x * x, axis=-1, keepdims=True)
    y2 = jnp.sum(y * y, axis=-1, keepdims=True)
    xy = jnp.sum(x * y, axis=-1, keepdims=True)
    num = (1.0 + 2.0 * xy + y2) * x + (1.0 - x2) * y
    denom = 1.0 + 2.0 * xy + x2 * y2
    return num / jnp.maximum(denom, _MIN_NORM)


def _mobius_matvec(W, x):
    xn = jnp.maximum(_norm(x), _MIN_NORM)
    mx = lax.dot_general(x, W, (((1,), (1,)), ((), ())),
                         precision=lax.Precision.HIGHEST)
    mxn = jnp.maximum(_norm(mx), _MIN_NORM)
    res = _tanh(mxn / xn * _artanh(xn)) * mx / mxn
    cond = jnp.all(mx == 0, axis=-1, keepdims=True)
    return jnp.where(cond, jnp.zeros_like(res), res)


def _hyp_linear_to_tangent(xh, W, b):
    res = _proj(_mobius_matvec(W, xh))
    hb = _proj(_expmap0(b))
    res = _proj(_mobius_add(res, hb))
    return _logmap0(res)


_BLK = 1000


def _k1_body(x_ref, w_ref, b_ref, o_ref):
    xh = _proj(_expmap0(x_ref[...]))
    o_ref[...] = _hyp_linear_to_tangent(xh, w_ref[...], b_ref[...])


def _k2_body(p_ref, w_ref, b_ref, o_ref):
    agg = p_ref[0] + p_ref[1]
    h = _proj(_expmap0(agg))
    xt = jax.nn.relu(_logmap0(h))
    xh = _proj(_expmap0(xt))
    o_ref[...] = _hyp_linear_to_tangent(xh, w_ref[...], b_ref[...])


def _k3_body(p_ref, o_ref):
    agg = p_ref[0] + p_ref[1]
    h = _proj(_expmap0(agg))
    xt = jax.nn.relu(_logmap0(h))
    o_ref[...] = _proj(_expmap0(xt))


def _dense_pre(x, W, b):
    return pl.pallas_call(
        _k1_body,
        grid=(N // _BLK,),
        in_specs=[
            pl.BlockSpec((_BLK, D), lambda i: (i, 0)),
            pl.BlockSpec((D, D), lambda i: (0, 0)),
            pl.BlockSpec((1, D), lambda i: (0, 0)),
        ],
        out_specs=pl.BlockSpec((_BLK, D), lambda i: (i, 0)),
        out_shape=jax.ShapeDtypeStruct((N, D), jnp.float32),
    )(x, W, b.reshape(1, D))


def _dense_mid(parts, W, b):
    return pl.pallas_call(
        _k2_body,
        grid=(N // _BLK,),
        in_specs=[
            pl.BlockSpec((2, _BLK, D), lambda i: (0, i, 0)),
            pl.BlockSpec((D, D), lambda i: (0, 0)),
            pl.BlockSpec((1, D), lambda i: (0, 0)),
        ],
        out_specs=pl.BlockSpec((_BLK, D), lambda i: (i, 0)),
        out_shape=jax.ShapeDtypeStruct((N, D), jnp.float32),
    )(parts, W, b.reshape(1, D))


def _dense_post(parts):
    return pl.pallas_call(
        _k3_body,
        grid=(N // _BLK,),
        in_specs=[
            pl.BlockSpec((2, _BLK, D), lambda i: (0, i, 0)),
        ],
        out_specs=pl.BlockSpec((_BLK, D), lambda i: (i, 0)),
        out_shape=jax.ShapeDtypeStruct((N, D), jnp.float32),
    )(parts)


def _lane_bcast(vec, j):
    return lax.gather(
        vec, jnp.full((16, 1), j, jnp.int32),
        lax.GatherDimensionNumbers(
            offset_dims=(), collapsed_slice_dims=(0,), start_index_map=(0,)),
        (1,), mode=lax.GatherScatterMode.PROMISE_IN_BOUNDS)


def _sc_body(xt_hbm, src_hbm, dst_hbm, w_hbm, out_hbm,
             src_v, dst_v, w_v, rows_v, zbuf_v, acc_sh,
             sem_i0, sem_i1, sem_i2, sem_i3,
             sem_g0, sem_g1, sem_g2, sem_g3,
             sem_a0, sem_a1, sem_a2, sem_a3):
    sem_i = (sem_i0, sem_i1, sem_i2, sem_i3)
    sem_g = (sem_g0, sem_g1, sem_g2, sem_g3)
    sem_a = (sem_a0, sem_a1, sem_a2, sem_a3)
    cid = lax.axis_index("c")
    sid = lax.axis_index("s")
    wid = cid * _NS + sid
    ebase = wid * _EPW
    row0 = sid * _RPT

    def issue_idx(c, b):
        base = ebase + c * _CHUNK
        pltpu.async_copy(src_hbm.at[pl.ds(base, _CHUNK)], src_v.at[b], sem_i[b])
        pltpu.async_copy(dst_hbm.at[pl.ds(base, _CHUNK)], dst_v.at[b], sem_i[b])
        pltpu.async_copy(w_hbm.at[pl.ds(base, _CHUNK)], w_v.at[b], sem_i[b])

    def wait_idx(b):
        pltpu.make_async_copy(
            src_hbm.at[pl.ds(0, _CHUNK)], src_v.at[b], sem_i[b]).wait()
        pltpu.make_async_copy(
            dst_hbm.at[pl.ds(0, _CHUNK)], dst_v.at[b], sem_i[b]).wait()
        pltpu.make_async_copy(
            w_hbm.at[pl.ds(0, _CHUNK)], w_v.at[b], sem_i[b]).wait()

    def issue_gather(b):
        pltpu.async_copy(xt_hbm.at[src_v.at[b]], rows_v.at[b], sem_g[b])

    def wait_gather(b):
        pltpu.make_async_copy(
            xt_hbm.at[src_v.at[b]], rows_v.at[b], sem_g[b]).wait()

    def issue_scatter(b):
        pltpu.async_copy(rows_v.at[b], acc_sh.at[dst_v.at[b]], sem_a[b],
                         add=True)

    def wait_scatter(b):
        pltpu.make_async_copy(
            rows_v.at[b], acc_sh.at[dst_v.at[b]], sem_a[b]).wait()

    def scale(b):
        @pl.loop(0, _CHUNK // 16)
        def _(g):
            wv = w_v[b, pl.ds(g * 16, 16)]

            @pl.loop(0, 16)
            def _(j):
                wb = _lane_bcast(wv, j)
                e = g * 16 + j
                for f in range(D // 16):
                    sl = pl.ds(f * 16, 16)
                    rows_v[b, e, sl] = rows_v[b, e, sl] * wb

    for b in range(4):
        issue_idx(b, b)

    @pl.loop(0, _ZROWS)
    def _(r):
        zero16 = jnp.zeros((16,), jnp.float32)
        for f in range(D // 16):
            zbuf_v[r, pl.ds(f * 16, 16)] = zero16

    zcopies = []
    for k in range(_RPT // _ZROWS):
        zcopies.append(pltpu.async_copy(
            zbuf_v, acc_sh.at[pl.ds(row0 + k * _ZROWS, _ZROWS)], sem_a0))
    for z in zcopies:
        z.wait()

    wait_idx(0)
    issue_gather(0)
    wait_idx(1)
    issue_gather(1)
    plsc.subcore_barrier()

    @pl.loop(0, _NCHUNK // 4)
    def _(h):
        for r in range(4):
            c = 4 * h + r

            def lookahead():
                wait_scatter((r + 3) % 4)
                issue_idx(c + 3, (r + 3) % 4)
                wait_idx((r + 2) % 4)
                issue_gather((r + 2) % 4)

            wait_gather(r)
            if r == 0:
                @pl.when(h >= 1)
                def _():
                    lookahead()
                @pl.when(h < 1)
                def _():
                    wait_idx(2)
                    issue_gather(2)
            elif r <= 1:
                lookahead()
            else:
                @pl.when(h <= _NCHUNK // 4 - 2)
                def _():
                    lookahead()
                @pl.when(h > _NCHUNK // 4 - 2)
                def _():
                    if r == 2:
                        wait_scatter((r + 3) % 4)
                        wait_idx((r + 2) % 4)
                        issue_gather((r + 2) % 4)
                    else:
                        wait_scatter((r + 3) % 4)
            scale(r)
            issue_scatter(r)

    wait_gather(0)
    scale(0)
    issue_scatter(0)
    wait_scatter(3)
    wait_scatter(0)
    plsc.subcore_barrier()

    drains = []
    for k in range(_RPT // 128):
        r = row0 + k * 128
        drains.append(pltpu.async_copy(
            acc_sh.at[pl.ds(r, 128)], out_hbm.at[cid, pl.ds(r, 128)],
            sem_g0))
    for d in drains:
        d.wait()


def _sc_aggregate(x_t, src, dst, w):
    kern = pl.kernel(
        _sc_body,
        out_type=jax.ShapeDtypeStruct((_NC, _NPAD, D), jnp.float32),
        mesh=plsc.VectorSubcoreMesh(core_axis_name="c", subcore_axis_name="s"),
        scratch_types=[
            pltpu.VMEM((4, _CHUNK), jnp.int32),
            pltpu.VMEM((4, _CHUNK), jnp.int32),
            pltpu.VMEM((4, _CHUNK), jnp.float32),
            pltpu.VMEM((4, _CHUNK, D), jnp.float32),
            pltpu.VMEM((_ZROWS, D), jnp.float32),
            pltpu.VMEM_SHARED((_NPAD, D), jnp.float32),
        ] + [pltpu.SemaphoreType.DMA] * 12,
    )
    return kern(x_t, src, dst, w)


def kernel(x, edge_index, edge_weight, W1, b1, W2, b2):
    src = edge_index[0]
    dst = edge_index[1]
    xt1 = _dense_pre(x, W1, b1)
    parts1 = _sc_aggregate(xt1, src, dst, edge_weight)
    xt2 = _dense_mid(parts1, W2, b2)
    parts2 = _sc_aggregate(xt2, src, dst, edge_weight)
    return _dense_post(parts2)


# speedup vs baseline: 1.7039x; 1.6927x over previous
import functools

import jax
import jax.numpy as jnp
from jax import lax
from jax.experimental import pallas as pl
from jax.experimental.pallas import tpu as pltpu
from jax.experimental.pallas import tpu_sc as plsc

N = 10000
E = 320000
D = 128

_MIN_NORM = 1e-15
_MAXNORM = 1.0 - 4e-3

_NC = 2
_NS = 16
_NW = _NC * _NS
_EPW = E // _NW
_CHUNK = 80
_NCHUNK = _EPW // _CHUNK
_NPAD = 10240
_RPT = _NPAD // _NS
_ZROWS = 16


def _norm(x):
    return jnp.sqrt(jnp.sum(x * x, axis=-1, keepdims=True))


def _artanh(x):
    x = jnp.clip(x, -1.0 + 1e-7, 1.0 - 1e-7)
    return 0.5 * jnp.log((1.0 + x) / (1.0 - x))


def _tanh(x):
    return jnp.tanh(jnp.clip(x, -15.0, 15.0))


def _proj(x):
    n = jnp.maximum(_norm(x), _MIN_NORM)
    return jnp.where(n > _MAXNORM, x / n * _MAXNORM, x)


def _expmap0(u):
    n = jnp.maximum(_norm(u), _MIN_NORM)
    return _tanh(n) * u / n


def _logmap0(p):
    n = jnp.maximum(_norm(p), _MIN_NORM)
    return p / n * _artanh(n)


def _mobius_add(x, y):
    x2 = jnp.sum(x * x, axis=-1, keepdims=True)
    y2 = jnp.sum(y * y, axis=-1, keepdims=True)
    xy = jnp.sum(x * y, axis=-1, keepdims=True)
    num = (1.0 + 2.0 * xy + y2) * x + (1.0 - x2) * y
    denom = 1.0 + 2.0 * xy + x2 * y2
    return num / jnp.maximum(denom, _MIN_NORM)


def _mobius_matvec(W, x):
    xn = jnp.maximum(_norm(x), _MIN_NORM)
    mx = lax.dot_general(x, W, (((1,), (1,)), ((), ())),
                         precision=lax.Precision.HIGHEST)
    mxn = jnp.maximum(_norm(mx), _MIN_NORM)
    res = _tanh(mxn / xn * _artanh(xn)) * mx / mxn
    cond = jnp.all(mx == 0, axis=-1, keepdims=True)
    return jnp.where(cond, jnp.zeros_like(res), res)


def _hyp_linear_to_tangent(xh, W, b):
    res = _proj(_mobius_matvec(W, xh))
    hb = _proj(_expmap0(b))
    res = _proj(_mobius_add(res, hb))
    return _logmap0(res)


_BLK = 1000


def _k1_body(x_ref, w_ref, b_ref, o_ref):
    xh = _proj(_expmap0(x_ref[...]))
    o_ref[...] = _hyp_linear_to_tangent(xh, w_ref[...], b_ref[...])


def _k2_body(p_ref, w_ref, b_ref, o_ref):
    agg = p_ref[0] + p_ref[1]
    h = _proj(_expmap0(agg))
    xt = jax.nn.relu(_logmap0(h))
    xh = _proj(_expmap0(xt))
    o_ref[...] = _hyp_linear_to_tangent(xh, w_ref[...], b_ref[...])


def _k3_body(p_ref, o_ref):
    agg = p_ref[0] + p_ref[1]
    h = _proj(_expmap0(agg))
    xt = jax.nn.relu(_logmap0(h))
    o_ref[...] = _proj(_expmap0(xt))


def _dense_pre(x, W, b):
    return pl.pallas_call(
        _k1_body,
        grid=(N // _BLK,),
        in_specs=[
            pl.BlockSpec((_BLK, D), lambda i: (i, 0)),
            pl.BlockSpec((D, D), lambda i: (0, 0)),
            pl.BlockSpec((1, D), lambda i: (0, 0)),
        ],
        out_specs=pl.BlockSpec((_BLK, D), lambda i: (i, 0)),
        out_shape=jax.ShapeDtypeStruct((N, D), jnp.float32),
    )(x, W, b.reshape(1, D))


def _dense_mid(parts, W, b):
    return pl.pallas_call(
        _k2_body,
        grid=(N // _BLK,),
        in_specs=[
            pl.BlockSpec((2, _BLK, D), lambda i: (0, i, 0)),
            pl.BlockSpec((D, D), lambda i: (0, 0)),
            pl.BlockSpec((1, D), lambda i: (0, 0)),
        ],
        out_specs=pl.BlockSpec((_BLK, D), lambda i: (i, 0)),
        out_shape=jax.ShapeDtypeStruct((N, D), jnp.float32),
    )(parts, W, b.reshape(1, D))


def _dense_post(parts):
    return pl.pallas_call(
        _k3_body,
        grid=(N // _BLK,),
        in_specs=[
            pl.BlockSpec((2, _BLK, D), lambda i: (0, i, 0)),
        ],
        out_specs=pl.BlockSpec((_BLK, D), lambda i: (i, 0)),
        out_shape=jax.ShapeDtypeStruct((N, D), jnp.float32),
    )(parts)


def _lane_bcast(vec, j):
    return lax.gather(
        vec, jnp.full((16, 1), j, jnp.int32),
        lax.GatherDimensionNumbers(
            offset_dims=(), collapsed_slice_dims=(0,), start_index_map=(0,)),
        (1,), mode=lax.GatherScatterMode.PROMISE_IN_BOUNDS)


def _sc_body(xt_hbm, src_hbm, dst_hbm, w_hbm, out_hbm,
             src_v, dst_v, w_v, rows_v, zbuf_v, acc_sh,
             sem_i0, sem_i1, sem_i2, sem_i3,
             sem_g0, sem_g1, sem_g2, sem_g3,
             sem_a0, sem_a1, sem_a2, sem_a3):
    sem_i = (sem_i0, sem_i1, sem_i2, sem_i3)
    sem_g = (sem_g0, sem_g1, sem_g2, sem_g3)
    sem_a = (sem_a0, sem_a1, sem_a2, sem_a3)
    cid = lax.axis_index("c")
    sid = lax.axis_index("s")
    wid = cid * _NS + sid
    ebase = wid * _EPW
    row0 = sid * _RPT

    def issue_idx(c, b):
        base = ebase + c * _CHUNK
        pltpu.async_copy(src_hbm.at[pl.ds(base, _CHUNK)], src_v.at[b], sem_i[b])
        pltpu.async_copy(dst_hbm.at[pl.ds(base, _CHUNK)], dst_v.at[b], sem_i[b])
        pltpu.async_copy(w_hbm.at[pl.ds(base, _CHUNK)], w_v.at[b], sem_i[b])

    def wait_idx(b):
        pltpu.make_async_copy(
            src_hbm.at[pl.ds(0, _CHUNK)], src_v.at[b], sem_i[b]).wait()
        pltpu.make_async_copy(
            dst_hbm.at[pl.ds(0, _CHUNK)], dst_v.at[b], sem_i[b]).wait()
        pltpu.make_async_copy(
            w_hbm.at[pl.ds(0, _CHUNK)], w_v.at[b], sem_i[b]).wait()

    def issue_gather(i, b):
        pltpu.async_copy(xt_hbm.at[src_v.at[i]], rows_v.at[b], sem_g[b])

    def wait_gather(i, b):
        pltpu.make_async_copy(
            xt_hbm.at[src_v.at[i]], rows_v.at[b], sem_g[b]).wait()

    def issue_scatter(i, b):
        pltpu.async_copy(rows_v.at[b], acc_sh.at[dst_v.at[i]], sem_a[b],
                         add=True)

    def wait_scatter(b):
        pltpu.make_async_copy(
            rows_v.at[b], acc_sh.at[dst_v.at[0]], sem_a[b]).wait()

    def scale(i, b):
        @pl.loop(0, _CHUNK // 16)
        def _(g):
            wv = w_v[i, pl.ds(g * 16, 16)]

            @pl.loop(0, 16)
            def _(j):
                wb = _lane_bcast(wv, j)
                e = g * 16 + j
                for f in range(D // 16):
                    sl = pl.ds(f * 16, 16)
                    rows_v[b, e, sl] = rows_v[b, e, sl] * wb

    issue_idx(0, 0)
    issue_idx(1, 1)

    @pl.loop(0, _ZROWS)
    def _(r):
        zero16 = jnp.zeros((16,), jnp.float32)
        for f in range(D // 16):
            zbuf_v[r, pl.ds(f * 16, 16)] = zero16

    zcopies = []
    for k in range(_RPT // _ZROWS):
        zcopies.append(pltpu.async_copy(
            zbuf_v, acc_sh.at[pl.ds(row0 + k * _ZROWS, _ZROWS)], sem_a0))
    for z in zcopies:
        z.wait()

    wait_idx(0)
    issue_gather(0, 0)
    plsc.subcore_barrier()

    @pl.loop(0, _NCHUNK // 4)
    def _(h):
        for r in range(4):
            c = 4 * h + r
            b = r % 2
            nb = 1 - b
            if r == 3:
                @pl.when(h <= _NCHUNK // 4 - 2)
                def _():
                    issue_idx(c + 2, (r + 2) % 4)
            else:
                issue_idx(c + 2, (r + 2) % 4)
            wait_gather(r, b)
            if r == 0:
                @pl.when(h >= 1)
                def _():
                    wait_scatter(nb)
            else:
                wait_scatter(nb)
            wait_idx((r + 1) % 4)
            issue_gather((r + 1) % 4, nb)
            scale(r, b)
            issue_scatter(r, b)

    wait_gather(0, 0)
    wait_scatter(1)
    scale(0, 0)
    issue_scatter(0, 0)
    wait_scatter(0)
    plsc.subcore_barrier()

    drains = []
    for k in range(_RPT // 128):
        r = row0 + k * 128
        drains.append(pltpu.async_copy(
            acc_sh.at[pl.ds(r, 128)], out_hbm.at[cid, pl.ds(r, 128)],
            sem_g0))
    for d in drains:
        d.wait()


def _sc_aggregate(x_t, src, dst, w):
    kern = pl.kernel(
        _sc_body,
        out_type=jax.ShapeDtypeStruct((_NC, _NPAD, D), jnp.float32),
        mesh=plsc.VectorSubcoreMesh(core_axis_name="c", subcore_axis_name="s"),
        scratch_types=[
            pltpu.VMEM((4, _CHUNK), jnp.int32),
            pltpu.VMEM((4, _CHUNK), jnp.int32),
            pltpu.VMEM((4, _CHUNK), jnp.float32),
            pltpu.VMEM((2, _CHUNK, D), jnp.float32),
            pltpu.VMEM((_ZROWS, D), jnp.float32),
            pltpu.VMEM_SHARED((_NPAD, D), jnp.float32),
        ] + [pltpu.SemaphoreType.DMA] * 12,
    )
    return kern(x_t, src, dst, w)


def kernel(x, edge_index, edge_weight, W1, b1, W2, b2):
    src = edge_index[0]
    dst = edge_index[1]
    xt1 = _dense_pre(x, W1, b1)
    parts1 = _sc_aggregate(xt1, src, dst, edge_weight)
    xt2 = _dense_mid(parts1, W2, b2)
    parts2 = _sc_aggregate(xt2, src, dst, edge_weight)
    return _dense_post(parts2)


# speedup vs baseline: 1.7074x; 1.0020x over previous
import functools

import jax
import jax.numpy as jnp
from jax import lax
from jax.experimental import pallas as pl
from jax.experimental.pallas import tpu as pltpu
from jax.experimental.pallas import tpu_sc as plsc

N = 10000
E = 320000
D = 128

_MIN_NORM = 1e-15
_MAXNORM = 1.0 - 4e-3

_NC = 2
_NS = 16
_NW = _NC * _NS
_EPW = E // _NW
_CHUNK = 80
_NCHUNK = _EPW // _CHUNK
_NPAD = 10240
_RPT = _NPAD // _NS
_ZROWS = 16


def _norm(x):
    return jnp.sqrt(jnp.sum(x * x, axis=-1, keepdims=True))


def _artanh(x):
    x = jnp.clip(x, -1.0 + 1e-7, 1.0 - 1e-7)
    return 0.5 * jnp.log((1.0 + x) / (1.0 - x))


def _tanh(x):
    return jnp.tanh(jnp.clip(x, -15.0, 15.0))


def _proj(x):
    n = jnp.maximum(_norm(x), _MIN_NORM)
    return jnp.where(n > _MAXNORM, x / n * _MAXNORM, x)


def _expmap0(u):
    n = jnp.maximum(_norm(u), _MIN_NORM)
    return _tanh(n) * u / n


def _logmap0(p):
    n = jnp.maximum(_norm(p), _MIN_NORM)
    return p / n * _artanh(n)


def _mobius_add(x, y):
    x2 = jnp.sum(x * x, axis=-1, keepdims=True)
    y2 = jnp.sum(y * y, axis=-1, keepdims=True)
    xy = jnp.sum(x * y, axis=-1, keepdims=True)
    num = (1.0 + 2.0 * xy + y2) * x + (1.0 - x2) * y
    denom = 1.0 + 2.0 * xy + x2 * y2
    return num / jnp.maximum(denom, _MIN_NORM)


def _mobius_matvec(W, x):
    xn = jnp.maximum(_norm(x), _MIN_NORM)
    mx = lax.dot_general(x, W, (((1,), (1,)), ((), ())),
                         precision=lax.Precision.HIGHEST)
    mxn = jnp.maximum(_norm(mx), _MIN_NORM)
    res = _tanh(mxn / xn * _artanh(xn)) * mx / mxn
    cond = jnp.all(mx == 0, axis=-1, keepdims=True)
    return jnp.where(cond, jnp.zeros_like(res), res)


def _hyp_linear_to_tangent(xh, W, b):
    res = _proj(_mobius_matvec(W, xh))
    hb = _proj(_expmap0(b))
    res = _proj(_mobius_add(res, hb))
    return _logmap0(res)


_BLK = 2000


def _k1_body(x_ref, w_ref, b_ref, o_ref):
    xh = _proj(_expmap0(x_ref[...]))
    o_ref[...] = _hyp_linear_to_tangent(xh, w_ref[...], b_ref[...])


def _k2_body(p_ref, w_ref, b_ref, o_ref):
    agg = p_ref[0] + p_ref[1]
    h = _proj(_expmap0(agg))
    xt = jax.nn.relu(_logmap0(h))
    xh = _proj(_expmap0(xt))
    o_ref[...] = _hyp_linear_to_tangent(xh, w_ref[...], b_ref[...])


def _k3_body(p_ref, o_ref):
    agg = p_ref[0] + p_ref[1]
    h = _proj(_expmap0(agg))
    xt = jax.nn.relu(_logmap0(h))
    o_ref[...] = _proj(_expmap0(xt))


def _dense_pre(x, W, b):
    return pl.pallas_call(
        _k1_body,
        grid=(N // _BLK,),
        in_specs=[
            pl.BlockSpec((_BLK, D), lambda i: (i, 0)),
            pl.BlockSpec((D, D), lambda i: (0, 0)),
            pl.BlockSpec((1, D), lambda i: (0, 0)),
        ],
        out_specs=pl.BlockSpec((_BLK, D), lambda i: (i, 0)),
        out_shape=jax.ShapeDtypeStruct((N, D), jnp.float32),
    )(x, W, b.reshape(1, D))


def _dense_mid(parts, W, b):
    return pl.pallas_call(
        _k2_body,
        grid=(N // _BLK,),
        in_specs=[
            pl.BlockSpec((2, _BLK, D), lambda i: (0, i, 0)),
            pl.BlockSpec((D, D), lambda i: (0, 0)),
            pl.BlockSpec((1, D), lambda i: (0, 0)),
        ],
        out_specs=pl.BlockSpec((_BLK, D), lambda i: (i, 0)),
        out_shape=jax.ShapeDtypeStruct((N, D), jnp.float32),
    )(parts, W, b.reshape(1, D))


def _dense_post(parts):
    return pl.pallas_call(
        _k3_body,
        grid=(N // _BLK,),
        in_specs=[
            pl.BlockSpec((2, _BLK, D), lambda i: (0, i, 0)),
        ],
        out_specs=pl.BlockSpec((_BLK, D), lambda i: (i, 0)),
        out_shape=jax.ShapeDtypeStruct((N, D), jnp.float32),
    )(parts)


def _lane_bcast(vec, j):
    return lax.gather(
        vec, jnp.full((16, 1), j, jnp.int32),
        lax.GatherDimensionNumbers(
            offset_dims=(), collapsed_slice_dims=(0,), start_index_map=(0,)),
        (1,), mode=lax.GatherScatterMode.PROMISE_IN_BOUNDS)


def _sc_body(xt_hbm, src_hbm, dst_hbm, w_hbm, out_hbm,
             src_v, dst_v, w_v, rows_v, zbuf_v, acc_sh,
             sem_i0, sem_i1, sem_i2, sem_i3,
             sem_g0, sem_g1, sem_g2, sem_g3,
             sem_a0, sem_a1, sem_a2, sem_a3):
    sem_i = (sem_i0, sem_i1, sem_i2, sem_i3)
    sem_g = (sem_g0, sem_g1, sem_g2, sem_g3)
    sem_a = (sem_a0, sem_a1, sem_a2, sem_a3)
    cid = lax.axis_index("c")
    sid = lax.axis_index("s")
    wid = cid * _NS + sid
    ebase = wid * _EPW
    row0 = sid * _RPT

    def issue_idx(c, b):
        base = ebase + c * _CHUNK
        pltpu.async_copy(src_hbm.at[pl.ds(base, _CHUNK)], src_v.at[b], sem_i[b])
        pltpu.async_copy(dst_hbm.at[pl.ds(base, _CHUNK)], dst_v.at[b], sem_i[b])

    def wait_idx(b):
        pltpu.make_async_copy(
            src_hbm.at[pl.ds(0, _CHUNK)], src_v.at[b], sem_i[b]).wait()
        pltpu.make_async_copy(
            dst_hbm.at[pl.ds(0, _CHUNK)], dst_v.at[b], sem_i[b]).wait()

    def issue_gather(i, b):
        pltpu.async_copy(xt_hbm.at[src_v.at[i]], rows_v.at[b], sem_g[b])

    def wait_gather(i, b):
        pltpu.make_async_copy(
            xt_hbm.at[src_v.at[i]], rows_v.at[b], sem_g[b]).wait()

    def issue_scatter(i, b):
        pltpu.async_copy(rows_v.at[b], acc_sh.at[dst_v.at[i]], sem_a[b],
                         add=True)

    def wait_scatter(b):
        pltpu.make_async_copy(
            rows_v.at[b], acc_sh.at[dst_v.at[0]], sem_a[b]).wait()

    def scale(c, b):
        @pl.loop(0, _CHUNK // 16)
        def _(g):
            wv = w_v[pl.ds(c * _CHUNK + g * 16, 16)]

            @pl.loop(0, 16)
            def _(j):
                wb = _lane_bcast(wv, j)
                e = g * 16 + j
                for f in range(D // 16):
                    sl = pl.ds(f * 16, 16)
                    rows_v[b, e, sl] = rows_v[b, e, sl] * wb

    wcopy = pltpu.async_copy(w_hbm.at[pl.ds(ebase, _EPW)], w_v, sem_g2)
    issue_idx(0, 0)
    issue_idx(1, 1)

    @pl.loop(0, _ZROWS)
    def _(r):
        zero16 = jnp.zeros((16,), jnp.float32)
        for f in range(D // 16):
            zbuf_v[r, pl.ds(f * 16, 16)] = zero16

    zcopies = []
    for k in range(_RPT // _ZROWS):
        zcopies.append(pltpu.async_copy(
            zbuf_v, acc_sh.at[pl.ds(row0 + k * _ZROWS, _ZROWS)], sem_a0))
    for z in zcopies:
        z.wait()

    wait_idx(0)
    issue_gather(0, 0)
    wcopy.wait()
    plsc.subcore_barrier()

    @pl.loop(0, _NCHUNK // 4)
    def _(h):
        for r in range(4):
            c = 4 * h + r
            b = r % 2
            nb = 1 - b
            if r == 3:
                @pl.when(h <= _NCHUNK // 4 - 2)
                def _():
                    issue_idx(c + 2, (r + 2) % 4)
            else:
                issue_idx(c + 2, (r + 2) % 4)
            wait_gather(r, b)
            if r == 0:
                @pl.when(h >= 1)
                def _():
                    wait_scatter(nb)
            else:
                wait_scatter(nb)
            wait_idx((r + 1) % 4)
            issue_gather((r + 1) % 4, nb)
            scale(c, b)
            issue_scatter(r, b)

    wait_gather(0, 0)
    wait_scatter(1)
    scale(_NCHUNK - 1, 0)
    issue_scatter(0, 0)
    wait_scatter(0)
    plsc.subcore_barrier()

    drains = []
    for k in range(_RPT // 128):
        r = row0 + k * 128
        drains.append(pltpu.async_copy(
            acc_sh.at[pl.ds(r, 128)], out_hbm.at[cid, pl.ds(r, 128)],
            sem_g0))
    for d in drains:
        d.wait()


def _sc_aggregate(x_t, src, dst, w):
    kern = pl.kernel(
        _sc_body,
        out_type=jax.ShapeDtypeStruct((_NC, _NPAD, D), jnp.float32),
        mesh=plsc.VectorSubcoreMesh(core_axis_name="c", subcore_axis_name="s"),
        scratch_types=[
            pltpu.VMEM((4, _CHUNK), jnp.int32),
            pltpu.VMEM((4, _CHUNK), jnp.int32),
            pltpu.VMEM((_EPW,), jnp.float32),
            pltpu.VMEM((2, _CHUNK, D), jnp.float32),
            pltpu.VMEM((_ZROWS, D), jnp.float32),
            pltpu.VMEM_SHARED((_NPAD, D), jnp.float32),
        ] + [pltpu.SemaphoreType.DMA] * 12,
    )
    return kern(x_t, src, dst, w)


def kernel(x, edge_index, edge_weight, W1, b1, W2, b2):
    src = edge_index[0]
    dst = edge_index[1]
    xt1 = _dense_pre(x, W1, b1)
    parts1 = _sc_aggregate(xt1, src, dst, edge_weight)
    xt2 = _dense_mid(parts1, W2, b2)
    parts2 = _sc_aggregate(xt2, src, dst, edge_weight)
    return _dense_post(parts2)
